# baseline (speedup 1.0000x reference)
import functools

import jax
import jax.numpy as jnp
from jax import lax
from jax.experimental import pallas as pl
from jax.experimental.pallas import tpu as pltpu
from jax.experimental.pallas import tpu_sc as plsc

N_NODES = 10000
N_EDGES = 320000
CHUNK = 64
RPW = 160
SB = 8
NW = 32
EDGES_PAD = NW * RPW * CHUNK
ACC_ROWS = 10016
RPT = ACC_ROWS // 16


def _make_sc_pass(width):
    gw = 128
    ngroups = gw // 16
    mesh = plsc.VectorSubcoreMesh(core_axis_name="c", subcore_axis_name="s")

    @functools.partial(
        pl.kernel,
        out_type=jax.ShapeDtypeStruct((2, ACC_ROWS, width), jnp.float32),
        mesh=mesh,
        scratch_types=[
            pltpu.VMEM((SB, CHUNK), jnp.int32),
            pltpu.VMEM((SB, CHUNK), jnp.int32),
            pltpu.VMEM((SB, CHUNK), jnp.float32),
            pltpu.VMEM((CHUNK, gw), jnp.float32),
            pltpu.VMEM((CHUNK, gw), jnp.float32),
            pltpu.VMEM((CHUNK, width), jnp.float32),
            pltpu.VMEM((CHUNK, width), jnp.float32),
            pltpu.VMEM_SHARED((ACC_ROWS, width), jnp.float32),
            pltpu.SemaphoreType.DMA,
            pltpu.SemaphoreType.DMA,
            pltpu.SemaphoreType.DMA,
            pltpu.SemaphoreType.DMA,
        ],
        compiler_params=pltpu.CompilerParams(use_tc_tiling_on_sc=False),
    )
    def sc_pass(x_hbm, src_hbm, dst_hbm, ew_hbm, out_hbm,
                src_v, dst_v, ew_v, g0, g1, s0, s1, acc,
                gsem0, gsem1, ssem0, ssem1):
        c = lax.axis_index("c")
        s = lax.axis_index("s")
        wid = c * 16 + s
        gbuf = (g0, g1)
        sbuf = (s0, s1)
        gsem = (gsem0, gsem1)
        ssem = (ssem0, ssem1)

        def g_start(t, b):
            pltpu.async_copy(x_hbm.at[src_v.at[t]], gbuf[b], gsem[b])

        def g_wait(t, b):
            pltpu.make_async_copy(
                x_hbm.at[src_v.at[t]], gbuf[b], gsem[b]).wait()

        def s_start(t, b):
            pltpu.async_copy(sbuf[b], acc.at[dst_v.at[t]], ssem[b], add=True)

        def s_wait(t, b):
            pltpu.make_async_copy(
                sbuf[b], acc.at[dst_v.at[t]], ssem[b]).wait()

        def scale(t, b):
            def escale(eb, _):
                wv = ew_v[t, pl.ds(eb * 16, 16)]
                for i in range(16):
                    w = wv[i]
                    e = eb * 16 + i
                    for g in range(ngroups):
                        sl = pl.ds(g * 16, 16)
                        sbuf[b][e, sl] = gbuf[b][e, sl] * w
                return 0
            lax.fori_loop(0, CHUNK // 16, escale, 0)

        def zrow(i, _):
            for g in range(width // 16):
                s0[i, pl.ds(g * 16, 16)] = jnp.zeros((16,), jnp.float32)
            return 0
        lax.fori_loop(0, CHUNK, zrow, 0)
        for k in range(RPT // CHUNK):
            pltpu.sync_copy(s0, acc.at[pl.ds(s * RPT + k * CHUNK, CHUNK)])
        pltpu.sync_copy(s0.at[pl.ds(0, RPT % CHUNK)],
                        acc.at[pl.ds(s * RPT + (RPT // CHUNK) * CHUNK,
                                     RPT % CHUNK)])
        if width > gw:
            def onesrow(i, _):
                for g in range(ngroups, width // 16):
                    sl = pl.ds(g * 16, 16)
                    s0[i, sl] = jnp.ones((16,), jnp.float32)
                    s1[i, sl] = jnp.ones((16,), jnp.float32)
                return 0
            lax.fori_loop(0, CHUNK, onesrow, 0)
        plsc.subcore_barrier()

        def sb_body(b, _):
            row0 = wid * RPW + b * SB
            pltpu.sync_copy(src_hbm.at[pl.ds(row0, SB)], src_v)
            pltpu.sync_copy(dst_hbm.at[pl.ds(row0, SB)], dst_v)
            pltpu.sync_copy(ew_hbm.at[pl.ds(row0, SB)], ew_v)

            g_start(0, 0)
            g_start(1, 1)

            def pair_body(p, _):
                for par in range(2):
                    t = 2 * p + par

                    @pl.when(p > 0)
                    def _():
                        s_wait(t - 2, par)
                    g_wait(t, par)
                    scale(t, par)

                    @pl.when(p < SB // 2 - 1)
                    def _():
                        g_start(t + 2, par)
                    s_start(t, par)
                return 0
            lax.fori_loop(0, SB // 2, pair_body, 0)
            s_wait(SB - 2, 0)
            s_wait(SB - 1, 1)
            return 0
        lax.fori_loop(0, RPW // SB, sb_body, 0)
        plsc.subcore_barrier()

        pltpu.sync_copy(acc.at[pl.ds(s * RPT, RPT)],
                        out_hbm.at[c, pl.ds(s * RPT, RPT)])

    return sc_pass


_sc_pass1 = _make_sc_pass(144)
_sc_pass2 = _make_sc_pass(128)


BN = 1000


def _tc1_body(x_ref, acc_ref, w1s_ref, w1n_ref, b1_ref, w2s_ref, w2n_ref,
              b2_ref, q_ref, s2_ref, inv_ref):
    a0 = acc_ref[0]
    a1 = acc_ref[1]
    deg = a0[:, 128:129] + a1[:, 128:129]
    inv = 1.0 / jnp.maximum(deg, 1.0)
    hn = (a0[:, :128] + a1[:, :128]) * inv
    h = jnp.dot(x_ref[...], w1s_ref[...], preferred_element_type=jnp.float32)
    h += jnp.dot(hn, w1n_ref[...], preferred_element_type=jnp.float32)
    h = jnp.maximum(h + b1_ref[...], 0.0)
    q_ref[...] = jnp.dot(h, w2n_ref[...], preferred_element_type=jnp.float32)
    s2_ref[...] = (jnp.dot(h, w2s_ref[...], preferred_element_type=jnp.float32)
                   + b2_ref[...])
    inv_ref[...] = inv


def _tc0_body(x_ref, o_ref):
    o_ref[...] = x_ref[...]


def _tc0(x):
    return pl.pallas_call(
        _tc0_body,
        grid=(N_NODES // BN,),
        in_specs=[pl.BlockSpec((BN, 128), lambda i: (i, 0))],
        out_specs=pl.BlockSpec((BN, 128), lambda i: (i, 0)),
        out_shape=jax.ShapeDtypeStruct((N_NODES, 128), jnp.float32),
    )(x)


def _tc2_body(s2_ref, inv_ref, acc_ref, o_ref):
    o_ref[...] = s2_ref[...] + inv_ref[...] * (acc_ref[0] + acc_ref[1])


def _tc1(x, acc1, w1s, w1n, b1, w2s, w2n, b2):
    grid = N_NODES // BN
    full = lambda shape: pl.BlockSpec(shape, lambda i: (0,) * len(shape))
    return pl.pallas_call(
        _tc1_body,
        grid=(grid,),
        in_specs=[
            pl.BlockSpec((BN, 128), lambda i: (i, 0)),
            pl.BlockSpec((2, BN, 144), lambda i: (0, i, 0)),
            full((128, 256)),
            full((128, 256)),
            full((1, 256)),
            full((256, 128)),
            full((256, 128)),
            full((1, 128)),
        ],
        out_specs=[
            pl.BlockSpec((BN, 128), lambda i: (i, 0)),
            pl.BlockSpec((BN, 128), lambda i: (i, 0)),
            pl.BlockSpec((BN, 1), lambda i: (i, 0)),
        ],
        out_shape=[
            jax.ShapeDtypeStruct((N_NODES, 128), jnp.float32),
            jax.ShapeDtypeStruct((N_NODES, 128), jnp.float32),
            jax.ShapeDtypeStruct((N_NODES, 1), jnp.float32),
        ],
    )(x, acc1, w1s, w1n, b1, w2s, w2n, b2)


def _tc2(s2, inv, acc2):
    grid = N_NODES // BN
    return pl.pallas_call(
        _tc2_body,
        grid=(grid,),
        in_specs=[
            pl.BlockSpec((BN, 128), lambda i: (i, 0)),
            pl.BlockSpec((BN, 1), lambda i: (i, 0)),
            pl.BlockSpec((2, BN, 128), lambda i: (0, i, 0)),
        ],
        out_specs=pl.BlockSpec((BN, 128), lambda i: (i, 0)),
        out_shape=jax.ShapeDtypeStruct((N_NODES, 128), jnp.float32),
    )(s2, inv, acc2)


@jax.jit
def kernel(edge_index, in_feat, edge_w, W1_self, W1_neigh, b1, W2_self,
           W2_neigh, b2):
    src = edge_index[0].astype(jnp.int32)
    dst = edge_index[1].astype(jnp.int32)
    pad = EDGES_PAD - N_EDGES
    pad_iota = jax.lax.iota(jnp.int32, pad)
    src_p = jnp.concatenate([src, pad_iota % N_NODES])
    src_p = src_p.reshape(NW * RPW, CHUNK)
    dst_p = jnp.concatenate([dst, N_NODES + pad_iota % (ACC_ROWS - N_NODES)])
    dst_p = dst_p.reshape(NW * RPW, CHUNK)
    ew_p = jnp.concatenate([edge_w, jnp.zeros((pad,), jnp.float32)])
    ew_p = ew_p.reshape(NW * RPW, CHUNK)

    x_lin = _tc0(in_feat)
    acc1 = _sc_pass1(x_lin, src_p, dst_p, ew_p)
    q, s2, inv = _tc1(x_lin, acc1, W1_self, W1_neigh, b1.reshape(1, -1),
                      W2_self, W2_neigh, b2.reshape(1, -1))
    acc2 = _sc_pass2(q, src_p, dst_p, ew_p)
    return _tc2(s2, inv, acc2)


# speedup vs baseline: 1.4740x; 1.4740x over previous
import functools

import jax
import jax.numpy as jnp
from jax import lax
from jax.experimental import pallas as pl
from jax.experimental.pallas import tpu as pltpu
from jax.experimental.pallas import tpu_sc as plsc

N_NODES = 10000
N_EDGES = 320000
CHUNK = 64
RPW = 160
NW = 32
EDGES_PAD = NW * RPW * CHUNK
ACC_ROWS = 10016
RPT = ACC_ROWS // 16


def _make_sc_pass(width, n_scaled, sb):
    ngroups = width // 16
    mesh = plsc.VectorSubcoreMesh(core_axis_name="c", subcore_axis_name="s")

    @functools.partial(
        pl.kernel,
        out_type=jax.ShapeDtypeStruct((2, ACC_ROWS, width), jnp.float32),
        mesh=mesh,
        scratch_types=[
            pltpu.VMEM((sb, CHUNK), jnp.int32),
            pltpu.VMEM((sb, CHUNK), jnp.int32),
            pltpu.VMEM((sb, CHUNK), jnp.float32),
            pltpu.VMEM((CHUNK, width), jnp.float32),
            pltpu.VMEM((CHUNK, width), jnp.float32),
            pltpu.VMEM((CHUNK, width), jnp.float32),
            pltpu.VMEM((CHUNK, width), jnp.float32),
            pltpu.VMEM_SHARED((ACC_ROWS, width), jnp.float32),
            pltpu.SemaphoreType.DMA,
            pltpu.SemaphoreType.DMA,
            pltpu.SemaphoreType.DMA,
            pltpu.SemaphoreType.DMA,
        ],
        compiler_params=pltpu.CompilerParams(use_tc_tiling_on_sc=False),
    )
    def sc_pass(x_hbm, src_hbm, dst_hbm, ew_hbm, out_hbm,
                src_v, dst_v, ew_v, g0, g1, s0, s1, acc,
                gsem0, gsem1, ssem0, ssem1):
        c = lax.axis_index("c")
        s = lax.axis_index("s")
        wid = c * 16 + s
        gbuf = (g0, g1)
        sbuf = (s0, s1)
        gsem = (gsem0, gsem1)
        ssem = (ssem0, ssem1)

        def g_start(t, b):
            pltpu.async_copy(x_hbm.at[src_v.at[t]], gbuf[b], gsem[b])

        def g_wait(t, b):
            pltpu.make_async_copy(
                x_hbm.at[src_v.at[t]], gbuf[b], gsem[b]).wait()

        def s_start(t, b):
            pltpu.async_copy(sbuf[b], acc.at[dst_v.at[t]], ssem[b], add=True)

        def s_wait(t, b):
            pltpu.make_async_copy(
                sbuf[b], acc.at[dst_v.at[t]], ssem[b]).wait()

        def scale(t, b):
            def escale(eb, _):
                wv = ew_v[t, pl.ds(eb * 16, 16)]
                for i in range(16):
                    w = wv[i]
                    e = eb * 16 + i
                    for g in range(ngroups):
                        sl = pl.ds(g * 16, 16)
                        if g < n_scaled:
                            sbuf[b][e, sl] = gbuf[b][e, sl] * w
                        else:
                            sbuf[b][e, sl] = gbuf[b][e, sl]
                return 0
            lax.fori_loop(0, CHUNK // 16, escale, 0)

        def zrow(i, _):
            for g in range(width // 16):
                s0[i, pl.ds(g * 16, 16)] = jnp.zeros((16,), jnp.float32)
            return 0
        lax.fori_loop(0, CHUNK, zrow, 0)
        for k in range(RPT // CHUNK):
            pltpu.sync_copy(s0, acc.at[pl.ds(s * RPT + k * CHUNK, CHUNK)])
        pltpu.sync_copy(s0.at[pl.ds(0, RPT % CHUNK)],
                        acc.at[pl.ds(s * RPT + (RPT // CHUNK) * CHUNK,
                                     RPT % CHUNK)])
        plsc.subcore_barrier()

        def sb_body(b, _):
            row0 = wid * RPW + b * sb
            pltpu.sync_copy(src_hbm.at[pl.ds(row0, sb)], src_v)
            pltpu.sync_copy(dst_hbm.at[pl.ds(row0, sb)], dst_v)
            pltpu.sync_copy(ew_hbm.at[pl.ds(row0, sb)], ew_v)

            g_start(0, 0)
            g_start(1, 1)

            def pair_body(p, _):
                for par in range(2):
                    t = 2 * p + par

                    @pl.when(p > 0)
                    def _():
                        s_wait(t - 2, par)
                    g_wait(t, par)
                    scale(t, par)

                    @pl.when(p < sb // 2 - 1)
                    def _():
                        g_start(t + 2, par)
                    s_start(t, par)
                return 0
            lax.fori_loop(0, sb // 2, pair_body, 0)
            s_wait(sb - 2, 0)
            s_wait(sb - 1, 1)
            return 0
        lax.fori_loop(0, RPW // sb, sb_body, 0)
        plsc.subcore_barrier()

        pltpu.sync_copy(acc.at[pl.ds(s * RPT, RPT)],
                        out_hbm.at[c, pl.ds(s * RPT, RPT)])

    return sc_pass


_sc_pass1 = _make_sc_pass(144, 8, 8)
_sc_pass2 = _make_sc_pass(128, 8, 16)


BN = 1000


def _tc1_body(x_ref, acc_ref, w1s_ref, w1n_ref, b1_ref, w2s_ref, w2n_ref,
              b2_ref, q_ref, s2_ref, inv_ref):
    a0 = acc_ref[0]
    a1 = acc_ref[1]
    deg = a0[:, 128:129] + a1[:, 128:129]
    inv = 1.0 / jnp.maximum(deg, 1.0)
    hn = (a0[:, :128] + a1[:, :128]) * inv
    h = jnp.dot(x_ref[...], w1s_ref[...], preferred_element_type=jnp.float32)
    h += jnp.dot(hn, w1n_ref[...], preferred_element_type=jnp.float32)
    h = jnp.maximum(h + b1_ref[...], 0.0)
    q_ref[...] = jnp.dot(h, w2n_ref[...], preferred_element_type=jnp.float32)
    s2_ref[...] = (jnp.dot(h, w2s_ref[...], preferred_element_type=jnp.float32)
                   + b2_ref[...])
    inv_ref[...] = inv


def _tc2_body(s2_ref, inv_ref, acc_ref, o_ref):
    o_ref[...] = s2_ref[...] + inv_ref[...] * (acc_ref[0] + acc_ref[1])


def _tc1(x, acc1, w1s, w1n, b1, w2s, w2n, b2):
    grid = N_NODES // BN
    full = lambda shape: pl.BlockSpec(shape, lambda i: (0,) * len(shape))
    return pl.pallas_call(
        _tc1_body,
        grid=(grid,),
        in_specs=[
            pl.BlockSpec((BN, 128), lambda i: (i, 0)),
            pl.BlockSpec((2, BN, 144), lambda i: (0, i, 0)),
            full((128, 256)),
            full((128, 256)),
            full((1, 256)),
            full((256, 128)),
            full((256, 128)),
            full((1, 128)),
        ],
        out_specs=[
            pl.BlockSpec((BN, 128), lambda i: (i, 0)),
            pl.BlockSpec((BN, 128), lambda i: (i, 0)),
            pl.BlockSpec((BN, 1), lambda i: (i, 0)),
        ],
        out_shape=[
            jax.ShapeDtypeStruct((N_NODES, 128), jnp.float32),
            jax.ShapeDtypeStruct((N_NODES, 128), jnp.float32),
            jax.ShapeDtypeStruct((N_NODES, 1), jnp.float32),
        ],
    )(x, acc1, w1s, w1n, b1, w2s, w2n, b2)


def _tc2(s2, inv, acc2):
    grid = N_NODES // BN
    return pl.pallas_call(
        _tc2_body,
        grid=(grid,),
        in_specs=[
            pl.BlockSpec((BN, 128), lambda i: (i, 0)),
            pl.BlockSpec((BN, 1), lambda i: (i, 0)),
            pl.BlockSpec((2, BN, 128), lambda i: (0, i, 0)),
        ],
        out_specs=pl.BlockSpec((BN, 128), lambda i: (i, 0)),
        out_shape=jax.ShapeDtypeStruct((N_NODES, 128), jnp.float32),
    )(s2, inv, acc2)


@jax.jit
def kernel(edge_index, in_feat, edge_w, W1_self, W1_neigh, b1, W2_self,
           W2_neigh, b2):
    src = edge_index[0].astype(jnp.int32)
    dst = edge_index[1].astype(jnp.int32)
    pad = EDGES_PAD - N_EDGES
    pad_iota = jax.lax.iota(jnp.int32, pad)
    src_p = jnp.concatenate([src, pad_iota % N_NODES])
    src_p = src_p.reshape(NW * RPW, CHUNK)
    dst_p = jnp.concatenate([dst, N_NODES + pad_iota % (ACC_ROWS - N_NODES)])
    dst_p = dst_p.reshape(NW * RPW, CHUNK)
    ew_p = jnp.concatenate([edge_w, jnp.zeros((pad,), jnp.float32)])
    ew_p = ew_p.reshape(NW * RPW, CHUNK)

    xw = jnp.concatenate(
        [in_feat, jnp.ones((N_NODES, 16), jnp.float32)], axis=1)
    acc1 = _sc_pass1(xw, src_p, dst_p, ew_p)
    q, s2, inv = _tc1(in_feat, acc1, W1_self, W1_neigh, b1.reshape(1, -1),
                      W2_self, W2_neigh, b2.reshape(1, -1))
    acc2 = _sc_pass2(q, src_p, dst_p, ew_p)
    return _tc2(s2, inv, acc2)


# speedup vs baseline: 1.5167x; 1.0290x over previous
import functools

import jax
import jax.numpy as jnp
from jax import lax
from jax.experimental import pallas as pl
from jax.experimental.pallas import tpu as pltpu
from jax.experimental.pallas import tpu_sc as plsc

N_NODES = 10000
N_EDGES = 320000
CHUNK = 64
RPW = 160
NW = 32
EDGES_PAD = NW * RPW * CHUNK
ACC_ROWS = 10016
RPT = ACC_ROWS // 16


def _make_sc_pass(width, n_scaled, sb):
    ngroups = width // 16
    mesh = plsc.VectorSubcoreMesh(core_axis_name="c", subcore_axis_name="s")

    @functools.partial(
        pl.kernel,
        out_type=jax.ShapeDtypeStruct((2, ACC_ROWS, width), jnp.float32),
        mesh=mesh,
        scratch_types=[
            pltpu.VMEM((sb, CHUNK), jnp.int32),
            pltpu.VMEM((sb, CHUNK), jnp.int32),
            pltpu.VMEM((sb, CHUNK), jnp.float32),
            pltpu.VMEM((CHUNK, width), jnp.float32),
            pltpu.VMEM((CHUNK, width), jnp.float32),
            pltpu.VMEM((CHUNK, width), jnp.float32),
            pltpu.VMEM((CHUNK, width), jnp.float32),
            pltpu.VMEM_SHARED((ACC_ROWS, width), jnp.float32),
            pltpu.SemaphoreType.DMA,
            pltpu.SemaphoreType.DMA,
            pltpu.SemaphoreType.DMA,
            pltpu.SemaphoreType.DMA,
        ],
        compiler_params=pltpu.CompilerParams(use_tc_tiling_on_sc=False),
    )
    def sc_pass(x_hbm, src_hbm, dst_hbm, ew_hbm, out_hbm,
                src_v, dst_v, ew_v, g0, g1, s0, s1, acc,
                gsem0, gsem1, ssem0, ssem1):
        c = lax.axis_index("c")
        s = lax.axis_index("s")
        wid = c * 16 + s
        gbuf = (g0, g1)
        sbuf = (s0, s1)
        gsem = (gsem0, gsem1)
        ssem = (ssem0, ssem1)

        def g_start(t, b):
            pltpu.async_copy(x_hbm.at[src_v.at[t]], gbuf[b], gsem[b])

        def g_wait(t, b):
            pltpu.make_async_copy(
                x_hbm.at[src_v.at[t]], gbuf[b], gsem[b]).wait()

        def s_start(t, b):
            pltpu.async_copy(sbuf[b], acc.at[dst_v.at[t]], ssem[b], add=True)

        def s_wait(t, b):
            pltpu.make_async_copy(
                sbuf[b], acc.at[dst_v.at[t]], ssem[b]).wait()

        def scale(t, b):
            def escale(eb, _):
                wv = ew_v[t, pl.ds(eb * 16, 16)]
                for i in range(16):
                    w = wv[i]
                    e = eb * 16 + i
                    for g in range(ngroups):
                        sl = pl.ds(g * 16, 16)
                        if g < n_scaled:
                            sbuf[b][e, sl] = gbuf[b][e, sl] * w
                        else:
                            sbuf[b][e, sl] = gbuf[b][e, sl]
                return 0
            lax.fori_loop(0, CHUNK // 16, escale, 0)

        def zrow(i, _):
            for g in range(width // 16):
                s0[i, pl.ds(g * 16, 16)] = jnp.zeros((16,), jnp.float32)
            return 0
        lax.fori_loop(0, CHUNK, zrow, 0)
        for k in range(RPT // CHUNK):
            pltpu.sync_copy(s0, acc.at[pl.ds(s * RPT + k * CHUNK, CHUNK)])
        pltpu.sync_copy(s0.at[pl.ds(0, RPT % CHUNK)],
                        acc.at[pl.ds(s * RPT + (RPT // CHUNK) * CHUNK,
                                     RPT % CHUNK)])
        plsc.subcore_barrier()

        def sb_body(b, _):
            row0 = wid * RPW + b * sb
            pltpu.sync_copy(src_hbm.at[pl.ds(row0, sb)], src_v)
            pltpu.sync_copy(dst_hbm.at[pl.ds(row0, sb)], dst_v)
            pltpu.sync_copy(ew_hbm.at[pl.ds(row0, sb)], ew_v)

            g_start(0, 0)
            g_start(1, 1)

            def pair_body(p, _):
                for par in range(2):
                    t = 2 * p + par

                    @pl.when(p > 0)
                    def _():
                        s_wait(t - 2, par)
                    g_wait(t, par)
                    scale(t, par)

                    @pl.when(p < sb // 2 - 1)
                    def _():
                        g_start(t + 2, par)
                    s_start(t, par)
                return 0
            lax.fori_loop(0, sb // 2, pair_body, 0)
            s_wait(sb - 2, 0)
            s_wait(sb - 1, 1)
            return 0
        lax.fori_loop(0, RPW // sb, sb_body, 0)
        plsc.subcore_barrier()

        pltpu.sync_copy(acc.at[pl.ds(s * RPT, RPT)],
                        out_hbm.at[c, pl.ds(s * RPT, RPT)])

    return sc_pass


_sc_pass1 = _make_sc_pass(144, 8, 8)
_sc_pass2 = _make_sc_pass(128, 8, 32)


BN = 1000


def _tc1_body(x_ref, acc_ref, w1s_ref, w1n_ref, b1_ref, w2s_ref, w2n_ref,
              b2_ref, q_ref, s2_ref, inv_ref):
    a0 = acc_ref[0]
    a1 = acc_ref[1]
    deg = a0[:, 128:129] + a1[:, 128:129]
    inv = 1.0 / jnp.maximum(deg, 1.0)
    hn = (a0[:, :128] + a1[:, :128]) * inv
    h = jnp.dot(x_ref[...], w1s_ref[...], preferred_element_type=jnp.float32)
    h += jnp.dot(hn, w1n_ref[...], preferred_element_type=jnp.float32)
    h = jnp.maximum(h + b1_ref[...], 0.0)
    q_ref[...] = jnp.dot(h, w2n_ref[...], preferred_element_type=jnp.float32)
    s2_ref[...] = (jnp.dot(h, w2s_ref[...], preferred_element_type=jnp.float32)
                   + b2_ref[...])
    inv_ref[...] = inv


def _tc2_body(s2_ref, inv_ref, acc_ref, o_ref):
    o_ref[...] = s2_ref[...] + inv_ref[...] * (acc_ref[0] + acc_ref[1])


def _tc1(x, acc1, w1s, w1n, b1, w2s, w2n, b2):
    grid = N_NODES // BN
    full = lambda shape: pl.BlockSpec(shape, lambda i: (0,) * len(shape))
    return pl.pallas_call(
        _tc1_body,
        grid=(grid,),
        in_specs=[
            pl.BlockSpec((BN, 128), lambda i: (i, 0)),
            pl.BlockSpec((2, BN, 144), lambda i: (0, i, 0)),
            full((128, 256)),
            full((128, 256)),
            full((1, 256)),
            full((256, 128)),
            full((256, 128)),
            full((1, 128)),
        ],
        out_specs=[
            pl.BlockSpec((BN, 128), lambda i: (i, 0)),
            pl.BlockSpec((BN, 128), lambda i: (i, 0)),
            pl.BlockSpec((BN, 1), lambda i: (i, 0)),
        ],
        out_shape=[
            jax.ShapeDtypeStruct((N_NODES, 128), jnp.float32),
            jax.ShapeDtypeStruct((N_NODES, 128), jnp.float32),
            jax.ShapeDtypeStruct((N_NODES, 1), jnp.float32),
        ],
    )(x, acc1, w1s, w1n, b1, w2s, w2n, b2)


def _tc2(s2, inv, acc2):
    grid = N_NODES // BN
    return pl.pallas_call(
        _tc2_body,
        grid=(grid,),
        in_specs=[
            pl.BlockSpec((BN, 128), lambda i: (i, 0)),
            pl.BlockSpec((BN, 1), lambda i: (i, 0)),
            pl.BlockSpec((2, BN, 128), lambda i: (0, i, 0)),
        ],
        out_specs=pl.BlockSpec((BN, 128), lambda i: (i, 0)),
        out_shape=jax.ShapeDtypeStruct((N_NODES, 128), jnp.float32),
    )(s2, inv, acc2)


@jax.jit
def kernel(edge_index, in_feat, edge_w, W1_self, W1_neigh, b1, W2_self,
           W2_neigh, b2):
    src = edge_index[0].astype(jnp.int32)
    dst = edge_index[1].astype(jnp.int32)
    pad = EDGES_PAD - N_EDGES
    pad_iota = jax.lax.iota(jnp.int32, pad)
    src_p = jnp.concatenate([src, pad_iota % N_NODES])
    src_p = src_p.reshape(NW * RPW, CHUNK)
    dst_p = jnp.concatenate([dst, N_NODES + pad_iota % (ACC_ROWS - N_NODES)])
    dst_p = dst_p.reshape(NW * RPW, CHUNK)
    ew_p = jnp.concatenate([edge_w, jnp.zeros((pad,), jnp.float32)])
    ew_p = ew_p.reshape(NW * RPW, CHUNK)

    xw = jnp.concatenate(
        [in_feat, jnp.ones((N_NODES, 16), jnp.float32)], axis=1)
    acc1 = _sc_pass1(xw, src_p, dst_p, ew_p)
    q, s2, inv = _tc1(in_feat, acc1, W1_self, W1_neigh, b1.reshape(1, -1),
                      W2_self, W2_neigh, b2.reshape(1, -1))
    acc2 = _sc_pass2(q, src_p, dst_p, ew_p)
    return _tc2(s2, inv, acc2)
